# baseline (speedup 1.0000x reference)
import functools

import jax
import jax.numpy as jnp
from jax import lax
from jax.experimental import pallas as pl
from jax.experimental.pallas import tpu as pltpu


def _logmatmul(a, b):
    ra = jnp.max(a, axis=2, keepdims=True)
    cb = jnp.max(b, axis=1, keepdims=True)
    ea = jnp.exp(a - ra)
    eb = jnp.exp(b - cb)
    p = lax.dot_general(ea, eb, (((2,), (1,)), ((0,), (0,))),
                        preferred_element_type=jnp.float32)
    return jnp.log(p) + ra + cb


def _halve(mats):
    n, c, _ = mats.shape
    pairs = mats.reshape(n // 2, 2, c, c)
    return _logmatmul(pairs[:, 0], pairs[:, 1])


def _decode_kernel(x_ref, dec_ref):
    x = x_ref[...]
    C = x.shape[1]
    mx = jnp.max(x, axis=1, keepdims=True)
    ids = lax.broadcasted_iota(jnp.int32, x.shape, 1)
    cand = jnp.where(x == mx, ids, jnp.int32(C))
    dec_ref[...] = jnp.min(cand, axis=1)


def _segment_kernel(emit_ref, trans_ref, seg_ref, *, levels):
    g = pl.program_id(0)
    emit = jnp.transpose(emit_ref[...], (2, 0, 1))
    tT, B, C = emit.shape

    tpos = lax.broadcasted_iota(jnp.int32, emit.shape, 0)
    first = jnp.logical_and(g == 0, tpos == 0)
    mats = emit + jnp.where(first, 0.0, trans_ref[...][None, :, :])

    for _ in range(levels):
        mats = _halve(mats)
    seg_ref[...] = mats


def _finish_kernel(seg_ref, nll_ref, *, levels, n_rem):
    mats = seg_ref[...]
    C = mats.shape[-1]
    for _ in range(levels):
        mats = _halve(mats)

    total = mats[0]
    if n_rem > 1:
        def fold(i, acc):
            nxt = mats[i]
            terms = [acc[:, m:m + 1] + nxt[m:m + 1, :] for m in range(C)]
            mx = terms[0]
            for t in terms[1:]:
                mx = jnp.maximum(mx, t)
            s = jnp.exp(terms[0] - mx)
            for t in terms[1:]:
                s = s + jnp.exp(t - mx)
            return jnp.log(s) + mx
        total = lax.fori_loop(1, n_rem, fold, total)

    m = jnp.max(total, axis=-1, keepdims=True)
    lse = jnp.log(jnp.sum(jnp.exp(total - m), axis=-1, keepdims=True)) + m
    nll_ref[...] = -jnp.sum(lse, axis=0, keepdims=True) / total.shape[0]


def _pow2_chunk(T, cap):
    c = 1
    while c < cap and T % (2 * c) == 0:
        c *= 2
    return c


def kernel(logits, mask, transitions):
    del mask
    B, C, H, W = logits.shape
    T = H * W

    logits_bct = logits.reshape(B, C, T).astype(jnp.float32)
    trans_t = jnp.swapaxes(transitions.astype(jnp.float32), 0, 1)

    tT = _pow2_chunk(T, 512)
    levels = tT.bit_length() - 1
    n_chunks = T // tT

    tC = _pow2_chunk(T, 2048)
    dec_bt = pl.pallas_call(
        _decode_kernel,
        out_shape=jax.ShapeDtypeStruct((B, T), jnp.int32),
        grid=(T // tC,),
        in_specs=[pl.BlockSpec((B, C, tC), lambda t: (0, 0, t))],
        out_specs=pl.BlockSpec((B, tC), lambda t: (0, t)),
        compiler_params=pltpu.CompilerParams(
            dimension_semantics=("parallel",),
        ),
    )(logits_bct)

    seg = pl.pallas_call(
        functools.partial(_segment_kernel, levels=levels),
        out_shape=jax.ShapeDtypeStruct((n_chunks, C, C), jnp.float32),
        grid=(n_chunks,),
        in_specs=[
            pl.BlockSpec((B, C, tT), lambda t: (0, 0, t)),
            pl.BlockSpec((C, C), lambda t: (0, 0)),
        ],
        out_specs=pl.BlockSpec((1, C, C), lambda t: (t, 0, 0)),
        compiler_params=pltpu.CompilerParams(
            dimension_semantics=("parallel",),
        ),
    )(logits_bct, trans_t)

    red_levels = 0
    n = n_chunks
    while n > 1 and n % 2 == 0:
        n //= 2
        red_levels += 1

    nll = pl.pallas_call(
        functools.partial(_finish_kernel, levels=red_levels, n_rem=n),
        out_shape=jax.ShapeDtypeStruct((1, 1), jnp.float32),
    )(seg)

    return nll[0, 0], dec_bt.reshape(B * T, 1)


# speedup vs baseline: 1.3668x; 1.3668x over previous
import functools

import jax
import jax.numpy as jnp
from jax import lax
from jax.experimental import pallas as pl
from jax.experimental.pallas import tpu as pltpu


def _logmatmul(a, b):
    ra = jnp.max(a, axis=2, keepdims=True)
    cb = jnp.max(b, axis=1, keepdims=True)
    ea = jnp.exp(a - ra)
    eb = jnp.exp(b - cb)
    p = lax.dot_general(ea, eb, (((2,), (1,)), ((0,), (0,))),
                        preferred_element_type=jnp.float32)
    return jnp.log(p) + ra + cb


def _halve(mats):
    n, c, _ = mats.shape
    pairs = mats.reshape(n // 2, 2, c, c)
    return _logmatmul(pairs[:, 0], pairs[:, 1])


def _decode_kernel(x_ref, dec_ref):
    x = x_ref[...]
    C = x.shape[1]
    mx = jnp.max(x, axis=1, keepdims=True)
    ids = lax.broadcasted_iota(jnp.int32, x.shape, 1)
    cand = jnp.where(x == mx, ids, jnp.int32(C))
    dec_ref[...] = jnp.min(cand, axis=1)


def _segment_kernel(emit_ref, trans_ref, seg_ref, *, levels):
    g = pl.program_id(0)
    emit = emit_ref[...]
    tT, B, C = emit.shape

    tpos = lax.broadcasted_iota(jnp.int32, emit.shape, 0)
    first = jnp.logical_and(g == 0, tpos == 0)
    mats = emit + jnp.where(first, 0.0, trans_ref[...][None, :, :])

    for _ in range(levels):
        mats = _halve(mats)
    seg_ref[...] = mats


def _finish_kernel(seg_ref, nll_ref, *, levels, n_rem):
    mats = seg_ref[...]
    C = mats.shape[-1]
    for _ in range(levels):
        mats = _halve(mats)

    total = mats[0]
    if n_rem > 1:
        def fold(i, acc):
            nxt = mats[i]
            terms = [acc[:, m:m + 1] + nxt[m:m + 1, :] for m in range(C)]
            mx = terms[0]
            for t in terms[1:]:
                mx = jnp.maximum(mx, t)
            s = jnp.exp(terms[0] - mx)
            for t in terms[1:]:
                s = s + jnp.exp(t - mx)
            return jnp.log(s) + mx
        total = lax.fori_loop(1, n_rem, fold, total)

    m = jnp.max(total, axis=-1, keepdims=True)
    lse = jnp.log(jnp.sum(jnp.exp(total - m), axis=-1, keepdims=True)) + m
    nll_ref[...] = -jnp.sum(lse, axis=0, keepdims=True) / total.shape[0]


def _pow2_chunk(T, cap):
    c = 1
    while c < cap and T % (2 * c) == 0:
        c *= 2
    return c


def kernel(logits, mask, transitions):
    del mask
    B, C, H, W = logits.shape
    T = H * W

    logits_bct = logits.reshape(B, C, T).astype(jnp.float32)
    logits_tbc = (jnp.transpose(logits, (2, 3, 0, 1))
                  .reshape(T, B, C).astype(jnp.float32))
    trans_t = jnp.swapaxes(transitions.astype(jnp.float32), 0, 1)

    tT = _pow2_chunk(T, 2048)
    levels = tT.bit_length() - 1
    n_chunks = T // tT

    tC = _pow2_chunk(T, 2048)
    dec_bt = pl.pallas_call(
        _decode_kernel,
        out_shape=jax.ShapeDtypeStruct((B, T), jnp.int32),
        grid=(T // tC,),
        in_specs=[pl.BlockSpec((B, C, tC), lambda t: (0, 0, t))],
        out_specs=pl.BlockSpec((B, tC), lambda t: (0, t)),
        compiler_params=pltpu.CompilerParams(
            dimension_semantics=("parallel",),
        ),
    )(logits_bct)

    seg = pl.pallas_call(
        functools.partial(_segment_kernel, levels=levels),
        out_shape=jax.ShapeDtypeStruct((n_chunks, C, C), jnp.float32),
        grid=(n_chunks,),
        in_specs=[
            pl.BlockSpec((tT, B, C), lambda t: (t, 0, 0)),
            pl.BlockSpec((C, C), lambda t: (0, 0)),
        ],
        out_specs=pl.BlockSpec((1, C, C), lambda t: (t, 0, 0)),
        compiler_params=pltpu.CompilerParams(
            dimension_semantics=("parallel",),
        ),
    )(logits_tbc, trans_t)

    red_levels = 0
    n = n_chunks
    while n > 1 and n % 2 == 0:
        n //= 2
        red_levels += 1

    nll = pl.pallas_call(
        functools.partial(_finish_kernel, levels=red_levels, n_rem=n),
        out_shape=jax.ShapeDtypeStruct((1, 1), jnp.float32),
    )(seg)

    return nll[0, 0], dec_bt.reshape(B * T, 1)
